# baseline (speedup 1.0000x reference)
import functools

import jax
import jax.numpy as jnp
from jax import lax
from jax.experimental import pallas as pl
from jax.experimental.pallas import tpu as pltpu
from jax.experimental.pallas import tpu_sc as plsc

_N = 10000
_E = 160000
_D = 256
_H = 128
_G = 64
_L = 4
_OUT = 11

_NC = 2
_NS = 16
_EP = 160256
_NT = 632
_OPAD = _NT * _NS
_NP = 10240
_CH = 128

_R = 2000
_NB = _N // _R


def _edge_body(z_hbm, sgidx_hbm, sdst_hbm, meta_hbm, zero_hbm, out_hbm,
               idx_v, lid_v, meta_v, rows_v, acc_v, sem):
    c = lax.axis_index("c")
    s = lax.axis_index("s")
    pltpu.sync_copy(zero_hbm, acc_v)
    pltpu.sync_copy(meta_hbm.at[pl.ds(pl.multiple_of(s * 16, 8), 16)], meta_v)
    mv = meta_v[pl.ds(0, 16)]
    a0 = mv[0]
    p0 = mv[1]
    p1 = mv[2]
    nch = (p1 - a0 + (_CH - 1)) // _CH
    zvec = jnp.zeros((16,), jnp.float32)

    def chunk(k, carry):
        off = pl.multiple_of(a0 + k * _CH, 8)
        pltpu.sync_copy(sgidx_hbm.at[pl.ds(pl.multiple_of(c * _EP + off, 8),
                                           _CH)], idx_v)
        pltpu.sync_copy(sdst_hbm.at[pl.ds(off, _CH)], lid_v)
        pltpu.async_copy(z_hbm.at[idx_v], rows_v, sem).wait()
        ilo = jnp.maximum(p0 - off, 0)
        ihi = jnp.minimum(p1 - off, _CH)

        def grp(g, gcarry):
            base_i = g * 16
            lvec = lid_v[pl.ds(pl.multiple_of(base_i, 8), 16)]
            for j in range(16):
                i = base_i + j
                d = lvec[j] - _NT * s
                d = jnp.minimum(jnp.maximum(d, 0), _NT - 1)
                ok = (i >= ilo) & (i < ihi)
                for k8 in range(8):
                    sl = pl.ds(k8 * 16, 16)
                    rv = jnp.where(ok, rows_v[i, sl], zvec)
                    acc_v[d, sl] = acc_v[d, sl] + rv
            return gcarry

        lax.fori_loop(0, _CH // 16, grp, 0)
        return carry

    lax.fori_loop(0, nch, chunk, 0)
    pltpu.sync_copy(acc_v,
                    out_hbm.at[pl.ds(pl.multiple_of(c * _OPAD + _NT * s, 8),
                                     _NT)])


@functools.cache
def _edge_kernel():
    return functools.partial(
        pl.kernel,
        out_type=jax.ShapeDtypeStruct((2 * _OPAD, _H), jnp.float32),
        mesh=plsc.VectorSubcoreMesh(core_axis_name="c", subcore_axis_name="s",
                                    num_cores=_NC, num_subcores=_NS),
        scratch_types=[
            pltpu.VMEM((_CH,), jnp.int32),
            pltpu.VMEM((_CH,), jnp.int32),
            pltpu.VMEM((16,), jnp.int32),
            pltpu.VMEM((_CH, _H), jnp.float32),
            pltpu.VMEM((_NT, _H), jnp.float32),
            pltpu.SemaphoreType.DMA,
        ],
    )(_edge_body)


def _edge_aggregate(z2d, sgidx, sdst, meta, zeros_nt):
    return _edge_kernel()(z2d, sgidx, sdst, meta, zeros_nt)


def _pool_body(hn_hbm, bat_hbm, meta_hbm, out_hbm,
               lid_v, meta_v, rows_v, acc_v):
    c = lax.axis_index("c")
    s = lax.axis_index("s")

    @pl.when(s < 8)
    def _():
        for r8 in range(8):
            for k8 in range(8):
                acc_v[r8, pl.ds(k8 * 16, 16)] = jnp.zeros((16,), jnp.float32)
        pltpu.sync_copy(meta_hbm.at[pl.ds(pl.multiple_of(s * 16, 8), 16)],
                        meta_v)
        mv = meta_v[pl.ds(0, 16)]
        a0 = mv[0]
        p0 = mv[1]
        p1 = mv[2]
        nch = (p1 - a0 + (_CH - 1)) // _CH
        zvec = jnp.zeros((16,), jnp.float32)

        def chunk(k, carry):
            off = pl.multiple_of(a0 + k * _CH, 8)
            pltpu.sync_copy(bat_hbm.at[pl.ds(off, _CH)], lid_v)
            pltpu.sync_copy(hn_hbm.at[pl.ds(off, _CH),
                                      pl.ds(pl.multiple_of(c * _H, 8), _H)],
                            rows_v)
            ilo = jnp.maximum(p0 - off, 0)
            ihi = jnp.minimum(p1 - off, _CH)

            def grp(g, gcarry):
                base_i = g * 16
                lvec = lid_v[pl.ds(pl.multiple_of(base_i, 8), 16)]
                for j in range(16):
                    i = base_i + j
                    d = lvec[j] - 8 * s
                    d = jnp.minimum(jnp.maximum(d, 0), 7)
                    ok = (i >= ilo) & (i < ihi)
                    for k8 in range(8):
                        sl = pl.ds(k8 * 16, 16)
                        rv = jnp.where(ok, rows_v[i, sl], zvec)
                        acc_v[d, sl] = acc_v[d, sl] + rv
                return gcarry

            lax.fori_loop(0, _CH // 16, grp, 0)
            return carry

        lax.fori_loop(0, nch, chunk, 0)
        pltpu.sync_copy(acc_v,
                       out_hbm.at[pl.ds(pl.multiple_of(c * _G + 8 * s, 8), 8)])


@functools.cache
def _pool_kernel():
    return functools.partial(
        pl.kernel,
        out_type=jax.ShapeDtypeStruct((2 * _G, _H), jnp.float32),
        mesh=plsc.VectorSubcoreMesh(core_axis_name="c", subcore_axis_name="s",
                                    num_cores=_NC, num_subcores=_NS),
        scratch_types=[
            pltpu.VMEM((_CH,), jnp.int32),
            pltpu.VMEM((16,), jnp.int32),
            pltpu.VMEM((_CH, _H), jnp.float32),
            pltpu.VMEM((8, _H), jnp.float32),
        ],
    )(_pool_body)


def _pool_sc(hn, bat_pad, meta2):
    return _pool_kernel()(hn, bat_pad, meta2)


def _write_z(z_ref, hv, tbl):
    for a in range(4):
        za = jnp.maximum(hv + tbl[a:a + 1, :], 0.0)
        for c in range(2):
            z_ref[c * 4 + a] = za[:, c * _H:(c + 1) * _H]


def _prep0_body(x_ref, nemb_ref, eemb_ref, lw_ref, lb_ref, hv_ref, z_ref):
    onehot = (x_ref[:, :] == lax.broadcasted_iota(jnp.int32, (_R, 32), 1)
              ).astype(jnp.float32)
    h = jnp.dot(onehot, nemb_ref[:, :], preferred_element_type=jnp.float32,
                precision=lax.Precision.HIGHEST)
    tbl = jnp.dot(eemb_ref[:, :], lw_ref[:, :],
                  preferred_element_type=jnp.float32) + lb_ref[:, :]
    hv_ref[:, :] = h
    _write_z(z_ref, h, tbl)


def _prep_body(h_ref, b_ref, vn_ref, eemb_ref, lw_ref, lb_ref, hv_ref, z_ref):
    onehot = (b_ref[:, :] == lax.broadcasted_iota(jnp.int32, (_R, _G), 1)
              ).astype(jnp.float32)
    hv = h_ref[:, :] + jnp.dot(onehot, vn_ref[:, :],
                               preferred_element_type=jnp.float32,
                               precision=lax.Precision.HIGHEST)
    tbl = jnp.dot(eemb_ref[:, :], lw_ref[:, :],
                  preferred_element_type=jnp.float32) + lb_ref[:, :]
    hv_ref[:, :] = hv
    _write_z(z_ref, hv, tbl)


def _mlp_body(hv_ref, ag0_ref, ag1_ref, w1_ref, b1_ref, w2_ref, b2_ref,
              y_ref, ps_ref):
    out = hv_ref[:, :] + jnp.concatenate([ag0_ref[0], ag1_ref[0]], axis=1)
    y = jnp.maximum(jnp.dot(out, w1_ref[:, :],
                            preferred_element_type=jnp.float32) + b1_ref[:, :],
                    0.0)
    y = jnp.dot(y, w2_ref[:, :], preferred_element_type=jnp.float32) \
        + b2_ref[:, :]
    y_ref[:, :] = y
    ps_ref[0] = jnp.sum(y, axis=0, keepdims=True)


def _var_body(y_ref, ps_ref, pv_ref):
    s = ps_ref[0]
    for i in range(1, _NB):
        s = s + ps_ref[i]
    mu = s / _N
    dy = y_ref[:, :] - mu
    pv_ref[0] = jnp.sum(dy * dy, axis=0, keepdims=True)


def _bnleaky_body(y_ref, ps_ref, pv_ref, g_ref, bb_ref, hn_ref):
    s = ps_ref[0]
    q = pv_ref[0]
    for i in range(1, _NB):
        s = s + ps_ref[i]
        q = q + pv_ref[i]
    mu = s / _N
    var = q / _N
    y = y_ref[:, :]
    yb = g_ref[:, :] * (y - mu) / jnp.sqrt(var + 1e-5) + bb_ref[:, :]
    hn_ref[:, :] = jnp.where(yb >= 0.0, yb, 0.1 * yb)


def _bn64(y, g, b, eps=1e-5):
    mu = jnp.mean(y, axis=0, keepdims=True)
    var = jnp.mean((y - mu) * (y - mu), axis=0, keepdims=True)
    return g * (y - mu) / jnp.sqrt(var + eps) + b


def _vnmlp_body(pool_ref, vn_ref, vw1_ref, vb1_ref, vg1_ref, vbe1_ref,
                vw2_ref, vb2_ref, vg2_ref, vbe2_ref, vn_out):
    pooled = jnp.concatenate([pool_ref[0:_G, :], pool_ref[_G:2 * _G, :]],
                             axis=1) + vn_ref[:, :]
    t = jnp.dot(pooled, vw1_ref[:, :], preferred_element_type=jnp.float32) \
        + vb1_ref[:, :]
    t = jnp.maximum(_bn64(t, vg1_ref[:, :], vbe1_ref[:, :]), 0.0)
    t = jnp.dot(t, vw2_ref[:, :], preferred_element_type=jnp.float32) \
        + vb2_ref[:, :]
    vn_out[:, :] = jnp.maximum(_bn64(t, vg2_ref[:, :], vbe2_ref[:, :]), 0.0)


def _head_body(hn_ref, bt_ref, hw1_ref, hb1_ref, hw2_ref, hb2_ref,
               hw3_ref, hb3_ref, o_ref):
    hn = hn_ref[0:_N, :]
    onehot_t = (lax.broadcasted_iota(jnp.int32, (_G, _N), 0) == bt_ref[:, :]
                ).astype(jnp.float32)
    pooled = jnp.dot(onehot_t, hn, preferred_element_type=jnp.float32,
                     precision=lax.Precision.HIGHEST)
    counts = jnp.dot(onehot_t, jnp.ones((_N, 1), jnp.float32),
                     preferred_element_type=jnp.float32,
                     precision=lax.Precision.HIGHEST)
    xg = pooled / jnp.maximum(counts, 1.0)
    o = jnp.maximum(jnp.dot(xg, hw1_ref[:, :],
                            preferred_element_type=jnp.float32) + hb1_ref[:, :],
                    0.0)
    o = jnp.maximum(jnp.dot(o, hw2_ref[:, :],
                            preferred_element_type=jnp.float32) + hb2_ref[:, :],
                    0.0)
    o_ref[:, :] = jnp.dot(o, hw3_ref[:, :],
                          preferred_element_type=jnp.float32) + hb3_ref[:, :]


def _full(shape):
    ix = tuple(0 for _ in shape)
    return pl.BlockSpec(shape, lambda i, _ix=ix: _ix)


_f32 = jnp.float32

_prep0 = pl.pallas_call(
    _prep0_body,
    grid=(_NB,),
    in_specs=[pl.BlockSpec((_R, 1), lambda i: (i, 0)),
              _full((32, _D)), _full((4, _D)), _full((_D, _D)),
              _full((1, _D))],
    out_specs=[pl.BlockSpec((_R, _D), lambda i: (i, 0)),
               pl.BlockSpec((8, _R, _H), lambda i: (0, i, 0))],
    out_shape=(jax.ShapeDtypeStruct((_N, _D), _f32),
               jax.ShapeDtypeStruct((8, _N, _H), _f32)),
)

_prep = pl.pallas_call(
    _prep_body,
    grid=(_NB,),
    in_specs=[pl.BlockSpec((_R, _D), lambda i: (i, 0)),
              pl.BlockSpec((_R, 1), lambda i: (i, 0)),
              _full((_G, _D)), _full((4, _D)), _full((_D, _D)),
              _full((1, _D))],
    out_specs=[pl.BlockSpec((_R, _D), lambda i: (i, 0)),
               pl.BlockSpec((8, _R, _H), lambda i: (0, i, 0))],
    out_shape=(jax.ShapeDtypeStruct((_N, _D), _f32),
               jax.ShapeDtypeStruct((8, _N, _H), _f32)),
)

_mlp = pl.pallas_call(
    _mlp_body,
    grid=(_NB,),
    in_specs=[pl.BlockSpec((_R, _D), lambda i: (i, 0)),
              pl.BlockSpec((1, _R, _H), lambda i: (0, i, 0)),
              pl.BlockSpec((1, _R, _H), lambda i: (1, i, 0)),
              _full((_D, _D)), _full((1, _D)), _full((_D, _D)),
              _full((1, _D))],
    out_specs=[pl.BlockSpec((_R, _D), lambda i: (i, 0)),
               pl.BlockSpec((1, 1, _D), lambda i: (i, 0, 0))],
    out_shape=(jax.ShapeDtypeStruct((_N, _D), _f32),
               jax.ShapeDtypeStruct((_NB, 1, _D), _f32)),
)

_var = pl.pallas_call(
    _var_body,
    grid=(_NB,),
    in_specs=[pl.BlockSpec((_R, _D), lambda i: (i, 0)),
              _full((_NB, 1, _D))],
    out_specs=pl.BlockSpec((1, 1, _D), lambda i: (i, 0, 0)),
    out_shape=jax.ShapeDtypeStruct((_NB, 1, _D), _f32),
)

_bnleaky = pl.pallas_call(
    _bnleaky_body,
    grid=(_NB,),
    in_specs=[pl.BlockSpec((_R, _D), lambda i: (i, 0)),
              _full((_NB, 1, _D)), _full((_NB, 1, _D)),
              _full((1, _D)), _full((1, _D))],
    out_specs=pl.BlockSpec((_R, _D), lambda i: (i, 0)),
    out_shape=jax.ShapeDtypeStruct((_NP, _D), _f32),
)

_vnmlp = pl.pallas_call(
    _vnmlp_body,
    out_shape=jax.ShapeDtypeStruct((_G, _D), _f32),
)

_head = pl.pallas_call(
    _head_body,
    out_shape=jax.ShapeDtypeStruct((_G, _OUT), _f32),
)


def kernel(x, edge_index, edge_attr, batch, node_emb, edge_emb, gine_lin_W,
           gine_lin_b, gine_W1, gine_b1, gine_W2, gine_b2, bn_g, bn_b,
           vn_W1, vn_b1, vn_g1, vn_be1, vn_W2, vn_b2, vn_g2, vn_be2,
           head_W1, head_b1, head_W2, head_b2, head_W3, head_b3):
    i32 = jnp.int32
    x2 = x.reshape(_N, 1).astype(i32)
    batch_col = batch.reshape(_N, 1).astype(i32)
    batch_row = batch.reshape(1, _N).astype(i32)
    src = edge_index[0].astype(i32)
    dst = edge_index[1].astype(i32)
    attr = edge_attr.astype(i32)

    perm = jnp.argsort(dst, stable=True)
    sdst = dst[perm]
    sg = (attr * _N + src)[perm]
    sg_pad = jnp.zeros((_EP,), i32).at[:_E].set(sg)
    sgidx = jnp.concatenate([sg_pad, sg_pad + 4 * _N])
    sdst_pad = jnp.zeros((_EP,), i32).at[:_E].set(sdst)
    starts = jnp.arange(_NS, dtype=i32) * _NT
    p0s = jnp.searchsorted(sdst, starts, side="left").astype(i32)
    p1s = jnp.append(p0s[1:], jnp.int32(_E)).astype(i32)
    a0s = (p0s // 8) * 8
    meta = (jnp.zeros((_NS, 16), i32).at[:, 0].set(a0s)
            .at[:, 1].set(p0s).at[:, 2].set(p1s)).reshape(-1)

    bat = batch.astype(i32)
    gstarts = jnp.arange(8, dtype=i32) * 8
    bp0 = jnp.searchsorted(bat, gstarts, side="left").astype(i32)
    bp1 = jnp.append(bp0[1:], jnp.int32(_N)).astype(i32)
    ba0 = (bp0 // 8) * 8
    meta2 = (jnp.zeros((_NS, 16), i32).at[:8, 0].set(ba0)
             .at[:8, 1].set(bp0).at[:8, 2].set(bp1)).reshape(-1)
    bat_pad = jnp.zeros((_NP,), i32).at[:_N].set(bat)

    zeros_nt = jnp.zeros((_NT, _H), _f32)
    nemb_p = jnp.zeros((32, _D), _f32).at[:28, :].set(node_emb)
    r = lambda v: v.reshape(1, -1)

    hv, z = _prep0(x2, nemb_p, edge_emb, gine_lin_W[0], r(gine_lin_b[0]))
    vn = jnp.zeros((_G, _D), _f32)
    for l in range(_L):
        aggr = _edge_aggregate(z.reshape(8 * _N, _H), sgidx, sdst_pad, meta,
                               zeros_nt)
        ag3 = aggr.reshape(2, _OPAD, _H)
        y, ps = _mlp(hv, ag3, ag3, gine_W1[l], r(gine_b1[l]),
                     gine_W2[l], r(gine_b2[l]))
        pv = _var(y, ps)
        hn = _bnleaky(y, ps, pv, r(bn_g[l]), r(bn_b[l]))
        if l < _L - 1:
            pool = _pool_sc(hn, bat_pad, meta2)
            vn = _vnmlp(pool, vn, vn_W1[l], r(vn_b1[l]), r(vn_g1[l]),
                        r(vn_be1[l]), vn_W2[l], r(vn_b2[l]), r(vn_g2[l]),
                        r(vn_be2[l]))
            hv, z = _prep(hn, batch_col, vn, edge_emb,
                          gine_lin_W[l + 1], r(gine_lin_b[l + 1]))
        else:
            o = _head(hn, batch_row, head_W1, r(head_b1), head_W2,
                      r(head_b2), head_W3, r(head_b3))
    return o
